# baseline (speedup 1.0000x reference)
import functools

import numpy as np
import jax
import jax.numpy as jnp
from jax import lax
from jax.experimental import pallas as pl
from jax.experimental.pallas import tpu as pltpu
from jax.experimental.pallas import tpu_sc as plsc

GRID_RES = (32, 64, 128, 256)
NUM_LOD = len(GRID_RES)
NC, NS = 2, 16
NW = NC * NS
N = 1000000
NP = 1 << 20
NROW = NP // 128
WROWS = NROW // NW
CH = 16
NCHUNK = WROWS // CH
BR = 256

_mesh = plsc.VectorSubcoreMesh(core_axis_name="c", subcore_axis_name="s")


def _idx_body(x_ref, y_ref, z_ref, o0_ref, o1_ref, o2_ref, o3_ref):
    hx = x_ref[...] * 0.5 + 0.5
    hy = y_ref[...] * 0.5 + 0.5
    hz = z_ref[...] * 0.5 + 0.5
    outs = (o0_ref, o1_ref, o2_ref, o3_ref)
    for l, res in enumerate(GRID_RES):
        s = np.float32(res - 1)
        ix = (hx * s).astype(jnp.int32)
        iy = (hy * s).astype(jnp.int32)
        iz = (hz * s).astype(jnp.int32)
        outs[l][...] = ix + iy * res + iz * (res * res)


_idx_tc = pl.pallas_call(
    _idx_body,
    grid=(NROW // BR,),
    in_specs=[pl.BlockSpec((BR, 128), lambda i: (i, 0))] * 3,
    out_specs=[pl.BlockSpec((BR, 128), lambda i: (i, 0))] * NUM_LOD,
    out_shape=[jax.ShapeDtypeStruct((NROW, 128), jnp.int32)] * NUM_LOD,
)


@functools.partial(
    pl.kernel,
    mesh=_mesh,
    out_type=jax.ShapeDtypeStruct((NP,), jnp.float32),
    scratch_types=[
        pltpu.VMEM((CH, 128), jnp.int32),
        pltpu.VMEM((NUM_LOD - 1, CH, 128), jnp.int32),
        pltpu.VMEM((NUM_LOD - 1, CH, 128), jnp.float32),
        pltpu.VMEM((CH * 128,), jnp.float32),
        pltpu.VMEM((GRID_RES[0] ** 3,), jnp.float32),
        pltpu.SemaphoreType.DMA,
    ],
    compiler_params=pltpu.CompilerParams(needs_layout_passes=False),
)
def _gather_sum(i0_hbm, i1_hbm, i2_hbm, i3_hbm, cb0_hbm, cb1_hbm, cb2_hbm,
                cb3_hbm, out_hbm, idx0_v, idx_v, feat_v, out_v, cb0_v, sem):
    cbs = (cb1_hbm, cb2_hbm, cb3_hbm)
    ihs = (i1_hbm, i2_hbm, i3_hbm)
    wid = lax.axis_index("s") * NC + lax.axis_index("c")
    pltpu.sync_copy(cb0_hbm, cb0_v)

    def chunk_body(t, carry):
        r0 = wid * WROWS + t * CH
        for l in range(NUM_LOD - 1):
            pltpu.async_copy(ihs[l].at[pl.ds(r0, CH)], idx_v.at[l], sem)
        pltpu.async_copy(i0_hbm.at[pl.ds(r0, CH)], idx0_v, sem)
        for l in range(NUM_LOD - 1):
            pltpu.make_async_copy(ihs[l].at[pl.ds(r0, CH)],
                                  idx_v.at[l], sem).wait()

        def fire(r, c2):
            for l, cb in enumerate(cbs):
                pltpu.async_copy(cb.at[idx_v.at[l, r]], feat_v.at[l, r], sem)
            return c2

        lax.fori_loop(0, CH, fire, 0)

        pltpu.make_async_copy(i0_hbm.at[pl.ds(r0, CH)], idx0_v, sem).wait()

        def lod0(r, c2):
            for u in range(128 // 16):
                idx = idx0_v[r, pl.ds(u * 16, 16)]
                out_v[pl.ds(r * 128 + u * 16, 16)] = \
                    plsc.load_gather(cb0_v, [idx])
            return c2

        lax.fori_loop(0, CH, lod0, 0)

        def drain_sum(r, c2):
            for l, cb in enumerate(cbs):
                pltpu.make_async_copy(cb.at[idx_v.at[l, r]],
                                      feat_v.at[l, r], sem).wait()
            for u in range(128 // 16):
                g = pl.ds(r * 128 + u * 16, 16)
                acc = out_v[g]
                for l in range(NUM_LOD - 1):
                    acc = acc + feat_v[l, r, pl.ds(u * 16, 16)]
                out_v[g] = acc
            return c2

        lax.fori_loop(0, CH, drain_sum, 0)
        pltpu.sync_copy(out_v, out_hbm.at[pl.ds(r0 * 128, CH * 128)])
        return carry

    lax.fori_loop(0, NCHUNK, chunk_body, 0)


def kernel(pts, cb0, cb1, cb2, cb3):
    pad = NP - N
    planes = [jnp.pad(pts[:, d], (0, pad)).reshape(NROW, 128)
              for d in range(3)]
    idx = _idx_tc(*planes)
    out = _gather_sum(*idx, cb0.reshape(-1), cb1.reshape(-1),
                      cb2.reshape(-1), cb3.reshape(-1))
    return out[:N, None]


# speedup vs baseline: 1.0372x; 1.0372x over previous
import functools

import numpy as np
import jax
import jax.numpy as jnp
from jax import lax
from jax.experimental import pallas as pl
from jax.experimental.pallas import tpu as pltpu
from jax.experimental.pallas import tpu_sc as plsc

GRID_RES = (32, 64, 128, 256)
NUM_LOD = len(GRID_RES)
NC, NS = 2, 16
NW = NC * NS
N = 1000000
NP = 1 << 20
WPTS = NP // NW
CP = 1024
CH = CP // 128
NCHUNK = WPTS // CP

_mesh = plsc.VectorSubcoreMesh(core_axis_name="c", subcore_axis_name="s")


@functools.partial(
    pl.kernel,
    mesh=_mesh,
    out_type=jax.ShapeDtypeStruct((NP,), jnp.float32),
    scratch_types=[
        pltpu.VMEM((2, CP), jnp.float32),
        pltpu.VMEM((2, CP), jnp.float32),
        pltpu.VMEM((2, CP), jnp.float32),
        pltpu.VMEM((2, NUM_LOD - 1, CH, 128), jnp.int32),
        pltpu.VMEM((2, NUM_LOD - 1, CH, 128), jnp.float32),
        pltpu.VMEM((2, CP), jnp.float32),
        pltpu.VMEM((GRID_RES[0] ** 3,), jnp.float32),
        pltpu.SemaphoreType.DMA,
    ],
    compiler_params=pltpu.CompilerParams(needs_layout_passes=False),
)
def _grid_gather(hx_hbm, hy_hbm, hz_hbm, cb0_hbm, cb1_hbm, cb2_hbm, cb3_hbm,
                 out_hbm, hx_v, hy_v, hz_v, idx_v, feat_v, out_v, cb0_v, sem):
    cbs = (cb1_hbm, cb2_hbm, cb3_hbm)
    hs = (hx_hbm, hy_hbm, hz_hbm)
    hvs = (hx_v, hy_v, hz_v)
    wid = lax.axis_index("s") * NC + lax.axis_index("c")
    base_w = wid * WPTS
    pltpu.sync_copy(cb0_hbm, cb0_v)

    def start_coords(t, p):
        base = base_w + t * CP
        for d in range(3):
            pltpu.async_copy(hs[d].at[pl.ds(base, CP)], hvs[d].at[p], sem)

    def wait_coords(t, p):
        base = base_w + t * CP
        for d in range(3):
            pltpu.make_async_copy(hs[d].at[pl.ds(base, CP)],
                                  hvs[d].at[p], sem).wait()

    def math_fire(p):
        def row_body(r, c2):
            for u in range(128 // 16):
                g = pl.ds(r * 128 + u * 16, 16)
                hx = hx_v[p, g]
                hy = hy_v[p, g]
                hz = hz_v[p, g]
                for l, res in enumerate(GRID_RES):
                    s = np.float32(res - 1)
                    ix = (hx * s).astype(jnp.int32)
                    iy = (hy * s).astype(jnp.int32)
                    iz = (hz * s).astype(jnp.int32)
                    idx = ix + iy * res + iz * (res * res)
                    if l == 0:
                        out_v[p, g] = plsc.load_gather(cb0_v, [idx])
                    else:
                        idx_v[p, l - 1, r, pl.ds(u * 16, 16)] = idx
            for l, cb in enumerate(cbs):
                pltpu.async_copy(cb.at[idx_v.at[p, l, r]],
                                 feat_v.at[p, l, r], sem)
            return c2

        lax.fori_loop(0, CH, row_body, 0)

    def drain_store(t, p):
        def row_body(r, c2):
            for l, cb in enumerate(cbs):
                pltpu.make_async_copy(cb.at[idx_v.at[p, l, r]],
                                      feat_v.at[p, l, r], sem).wait()
            for u in range(128 // 16):
                g = pl.ds(r * 128 + u * 16, 16)
                acc = out_v[p, g]
                for l in range(NUM_LOD - 1):
                    acc = acc + feat_v[p, l, r, pl.ds(u * 16, 16)]
                out_v[p, g] = acc
            return c2

        lax.fori_loop(0, CH, row_body, 0)
        pltpu.sync_copy(out_v.at[p], out_hbm.at[pl.ds(base_w + t * CP, CP)])

    start_coords(0, 0)

    def chunk_body(t, carry):
        p = t % 2
        wait_coords(t, p)

        @pl.when(t + 1 < NCHUNK)
        def _():
            start_coords(t + 1, 1 - p)

        math_fire(p)

        @pl.when(t >= 1)
        def _():
            drain_store(t - 1, 1 - p)

        return carry

    lax.fori_loop(0, NCHUNK, chunk_body, 0)
    drain_store(NCHUNK - 1, (NCHUNK - 1) % 2)


def kernel(pts, cb0, cb1, cb2, cb3):
    pad = NP - N
    hx = jnp.pad(pts[:, 0] * 0.5 + 0.5, (0, pad))
    hy = jnp.pad(pts[:, 1] * 0.5 + 0.5, (0, pad))
    hz = jnp.pad(pts[:, 2] * 0.5 + 0.5, (0, pad))
    out = _grid_gather(hx, hy, hz, cb0.reshape(-1), cb1.reshape(-1),
                       cb2.reshape(-1), cb3.reshape(-1))
    return out[:N, None]
